# Initial kernel scaffold; baseline (speedup 1.0000x reference)
import jax
import jax.numpy as jnp
from jax.experimental import pallas as pl


def kernel(x, edge_index, W1, b1, W2, b2):
    raise NotImplementedError("write your pallas kernel here")


# speedup vs baseline: 14.5809x; 14.5809x over previous
import functools

import jax
import jax.numpy as jnp
from jax import lax
from jax.experimental import pallas as pl
from jax.experimental.pallas import tpu as pltpu
from jax.experimental.pallas import tpu_sc as plsc

N = 10000
NPAD = 10240
D = 128
DO = 16
NC, NS = 2, 16
NW = NC * NS
CH = 128
BR = 1024

_MESH = plsc.VectorSubcoreMesh(core_axis_name="c", subcore_axis_name="s")
_SC_PARAMS = pltpu.CompilerParams(use_tc_tiling_on_sc=False)


def _zero_vmem_2d(ref, rows, cols):
    zc = cols // 16

    def body(i, _):
        r = i // zc
        k = i % zc
        ref[r, pl.ds(k * 16, 16)] = jnp.zeros((16,), jnp.float32)
        return 0

    lax.fori_loop(0, rows * zc, body, 0)


def _make_deg_kernel(ept):
    n_chunks = ept // CH

    @functools.partial(
        pl.kernel,
        out_type=jax.ShapeDtypeStruct((NC * NPAD,), jnp.float32),
        mesh=_MESH,
        compiler_params=_SC_PARAMS,
        scratch_types=[
            pltpu.VMEM((CH,), jnp.int32),
            pltpu.VMEM((CH,), jnp.float32),
            pltpu.VMEM((NPAD // NS,), jnp.float32),
            pltpu.VMEM_SHARED((NPAD,), jnp.float32),
            pltpu.SemaphoreType.DMA,
        ],
    )
    def deg_kernel(dst_hbm, deg_hbm, idx_v, ones_v, stage_v, acc_sh, sem):
        c = lax.axis_index("c")
        s = lax.axis_index("s")
        wid = c * NS + s
        seg = NPAD // NS

        def zbody(i, _):
            stage_v[pl.ds(i * 16, 16)] = jnp.zeros((16,), jnp.float32)
            return 0

        lax.fori_loop(0, seg // 16, zbody, 0)

        def obody(i, _):
            ones_v[pl.ds(i * 16, 16)] = jnp.ones((16,), jnp.float32)
            return 0

        lax.fori_loop(0, CH // 16, obody, 0)

        pltpu.sync_copy(stage_v, acc_sh.at[pl.ds(s * seg, seg)])
        plsc.subcore_barrier()

        ebase = wid * ept

        def chunk(i, _):
            pltpu.sync_copy(dst_hbm.at[pl.ds(ebase + i * CH, CH)], idx_v)
            pltpu.sync_copy(ones_v, acc_sh.at[idx_v], add=True)
            return 0

        lax.fori_loop(0, n_chunks, chunk, 0)
        plsc.subcore_barrier()

        pltpu.sync_copy(acc_sh.at[pl.ds(s * seg, seg)], stage_v)
        pltpu.sync_copy(stage_v, deg_hbm.at[pl.ds(c * NPAD + s * seg, seg)])

    return deg_kernel


def _make_scatter_kernel(ept, fd):
    n_chunks = ept // CH
    rows_per_tile = NPAD // NW
    seg = NPAD // NS

    @functools.partial(
        pl.kernel,
        out_type=jax.ShapeDtypeStruct((NC * NPAD, fd), jnp.float32),
        mesh=_MESH,
        compiler_params=_SC_PARAMS,
        scratch_types=[
            pltpu.VMEM((CH,), jnp.int32),
            pltpu.VMEM((CH,), jnp.int32),
            pltpu.VMEM((CH, fd), jnp.float32),
            pltpu.VMEM_SHARED((NPAD, fd), jnp.float32),
            pltpu.SemaphoreType.DMA,
        ],
    )
    def scat_kernel(src_hbm, dst_hbm, feat_hbm, out_hbm,
                    si_v, di_v, rows_v, acc_sh, sem):
        c = lax.axis_index("c")
        s = lax.axis_index("s")
        wid = c * NS + s

        _zero_vmem_2d(rows_v, CH, fd)
        nz = seg // CH

        def zc(i, _):
            pltpu.sync_copy(rows_v, acc_sh.at[pl.ds(s * seg + i * CH, CH)])
            return 0

        lax.fori_loop(0, nz, zc, 0)
        plsc.subcore_barrier()

        ebase = wid * ept

        def chunk(i, _):
            pltpu.sync_copy(src_hbm.at[pl.ds(ebase + i * CH, CH)], si_v)
            pltpu.sync_copy(dst_hbm.at[pl.ds(ebase + i * CH, CH)], di_v)
            pltpu.async_copy(feat_hbm.at[si_v], rows_v, sem).wait()
            pltpu.sync_copy(rows_v, acc_sh.at[di_v], add=True)
            return 0

        lax.fori_loop(0, n_chunks, chunk, 0)
        plsc.subcore_barrier()

        def oc(i, _):
            pltpu.sync_copy(acc_sh.at[pl.ds(s * seg + i * CH, CH)], rows_v)
            pltpu.sync_copy(
                rows_v, out_hbm.at[pl.ds(c * NPAD + s * seg + i * CH, CH)])
            return 0

        lax.fori_loop(0, nz, oc, 0)

    return scat_kernel


def _tc_b(deg0_ref, deg1_ref, x_ref, w1_ref, hs_ref, dinv_ref):
    deg = deg0_ref[...] + deg1_ref[...] + 1.0
    dinv = lax.rsqrt(deg)
    h = jnp.dot(x_ref[...], w1_ref[...], preferred_element_type=jnp.float32)
    hs_ref[...] = h * dinv
    dinv_ref[...] = dinv


def _tc_d(acc0_ref, acc1_ref, hs_ref, dinv_ref, b1_ref, w2_ref, gs_ref):
    i = pl.program_id(0)
    dinv = dinv_ref[...]
    pre = acc0_ref[...] + acc1_ref[...] + hs_ref[...]
    o = pre * dinv + b1_ref[...]
    h1 = jnp.maximum(o, 0.0)
    g = jnp.dot(h1, w2_ref[...], preferred_element_type=jnp.float32)
    row = jax.lax.broadcasted_iota(jnp.int32, (BR, 1), 0) + i * BR
    gs_ref[...] = jnp.where(row < N, g * dinv, 0.0)


def _tc_f(a0_ref, a1_ref, gs_ref, dinv_ref, b2_ref, out_ref):
    o = (a0_ref[...] + a1_ref[...] + gs_ref[...]) * dinv_ref[...] + b2_ref[...]
    o8 = o[:, :8]
    m = jnp.max(o8, axis=1, keepdims=True)
    e = jnp.exp(o8 - m)
    lse = jnp.log(jnp.sum(e, axis=1, keepdims=True))
    out_ref[...] = o8 - m - lse


def kernel(x, edge_index, W1, b1, W2, b2):
    E = edge_index.shape[1]
    ept = ((E + NW * CH - 1) // (NW * CH)) * CH
    epad = ept * NW
    pad = epad - E

    src = jnp.concatenate(
        [edge_index[0], jnp.full((pad,), N, jnp.int32)])
    dst = jnp.concatenate(
        [edge_index[1], jnp.full((pad,), N, jnp.int32)])

    x_pad = jnp.pad(x, ((0, NPAD - N), (0, 0)))
    w2p = jnp.pad(W2, ((0, 0), (0, DO - W2.shape[1])))
    b1r = b1.reshape(1, D)
    b2r = jnp.pad(b2, (0, DO - b2.shape[0])).reshape(1, DO)

    deg = _make_deg_kernel(ept)(dst)
    deg0 = deg[:NPAD].reshape(NPAD, 1)
    deg1 = deg[NPAD:].reshape(NPAD, 1)

    grid = NPAD // BR
    hs, dinv = pl.pallas_call(
        _tc_b,
        grid=(grid,),
        in_specs=[
            pl.BlockSpec((BR, 1), lambda i: (i, 0)),
            pl.BlockSpec((BR, 1), lambda i: (i, 0)),
            pl.BlockSpec((BR, D), lambda i: (i, 0)),
            pl.BlockSpec((D, D), lambda i: (0, 0)),
        ],
        out_specs=[
            pl.BlockSpec((BR, D), lambda i: (i, 0)),
            pl.BlockSpec((BR, 1), lambda i: (i, 0)),
        ],
        out_shape=[
            jax.ShapeDtypeStruct((NPAD, D), jnp.float32),
            jax.ShapeDtypeStruct((NPAD, 1), jnp.float32),
        ],
    )(deg0, deg1, x_pad, W1)

    acc = _make_scatter_kernel(ept, D)(src, dst, hs)
    acc0 = acc[:NPAD]
    acc1 = acc[NPAD:]

    gs = pl.pallas_call(
        _tc_d,
        grid=(grid,),
        in_specs=[
            pl.BlockSpec((BR, D), lambda i: (i, 0)),
            pl.BlockSpec((BR, D), lambda i: (i, 0)),
            pl.BlockSpec((BR, D), lambda i: (i, 0)),
            pl.BlockSpec((BR, 1), lambda i: (i, 0)),
            pl.BlockSpec((1, D), lambda i: (0, 0)),
            pl.BlockSpec((D, DO), lambda i: (0, 0)),
        ],
        out_specs=pl.BlockSpec((BR, DO), lambda i: (i, 0)),
        out_shape=jax.ShapeDtypeStruct((NPAD, DO), jnp.float32),
    )(acc0, acc1, hs, dinv, b1r, w2p)

    acc2 = _make_scatter_kernel(ept, DO)(src, dst, gs)
    a20 = acc2[:NPAD]
    a21 = acc2[NPAD:]

    out = pl.pallas_call(
        _tc_f,
        grid=(grid,),
        in_specs=[
            pl.BlockSpec((BR, DO), lambda i: (i, 0)),
            pl.BlockSpec((BR, DO), lambda i: (i, 0)),
            pl.BlockSpec((BR, DO), lambda i: (i, 0)),
            pl.BlockSpec((BR, 1), lambda i: (i, 0)),
            pl.BlockSpec((1, DO), lambda i: (0, 0)),
        ],
        out_specs=pl.BlockSpec((BR, 8), lambda i: (i, 0)),
        out_shape=jax.ShapeDtypeStruct((NPAD, 8), jnp.float32),
    )(a20, a21, gs, dinv, b2r)

    return out[:N]


# speedup vs baseline: 17.2713x; 1.1845x over previous
import functools

import jax
import jax.numpy as jnp
from jax import lax
from jax.experimental import pallas as pl
from jax.experimental.pallas import tpu as pltpu
from jax.experimental.pallas import tpu_sc as plsc

N = 10000
NPAD = 10240
D = 128
DO = 16
NC, NS = 2, 16
NW = NC * NS
CH = 128
BR = 1024

_MESH = plsc.VectorSubcoreMesh(core_axis_name="c", subcore_axis_name="s")
_SC_PARAMS = pltpu.CompilerParams(use_tc_tiling_on_sc=False)


def _zero_vmem_2d(ref, rows, cols):
    zc = cols // 16

    def body(i, _):
        r = i // zc
        k = i % zc
        ref[r, pl.ds(k * 16, 16)] = jnp.zeros((16,), jnp.float32)
        return 0

    lax.fori_loop(0, rows * zc, body, 0)


def _make_deg_kernel(ept):
    n_chunks = ept // CH
    grp = 16

    @functools.partial(
        pl.kernel,
        out_type=jax.ShapeDtypeStruct((NC * NPAD,), jnp.float32),
        mesh=_MESH,
        compiler_params=_SC_PARAMS,
        scratch_types=[
            pltpu.VMEM((n_chunks, CH), jnp.int32),
            pltpu.VMEM((CH,), jnp.float32),
            pltpu.VMEM((NPAD // NS,), jnp.float32),
            pltpu.VMEM_SHARED((NPAD,), jnp.float32),
            pltpu.SemaphoreType.DMA,
            pltpu.SemaphoreType.DMA,
        ],
    )
    def deg_kernel(dst_hbm, deg_hbm, idx_v, ones_v, stage_v, acc_sh,
                   sem_i, sem_s):
        c = lax.axis_index("c")
        s = lax.axis_index("s")
        wid = c * NS + s
        seg = NPAD // NS

        idx_dma = pltpu.async_copy(
            dst_hbm.at[pl.ds(wid * n_chunks, n_chunks)], idx_v, sem_i)

        def zbody(i, _):
            stage_v[pl.ds(i * 16, 16)] = jnp.zeros((16,), jnp.float32)
            return 0

        lax.fori_loop(0, seg // 16, zbody, 0)

        def obody(i, _):
            ones_v[pl.ds(i * 16, 16)] = jnp.ones((16,), jnp.float32)
            return 0

        lax.fori_loop(0, CH // 16, obody, 0)

        pltpu.sync_copy(stage_v, acc_sh.at[pl.ds(s * seg, seg)])
        plsc.subcore_barrier()
        idx_dma.wait()

        def group(g, _):
            def fire(j, _):
                pltpu.async_copy(
                    ones_v, acc_sh.at[idx_v.at[g * grp + j]], sem_s, add=True)
                return 0

            lax.fori_loop(0, grp, fire, 0)

            def drain(j, _):
                pltpu.make_async_copy(
                    ones_v, acc_sh.at[idx_v.at[0]], sem_s).wait()
                return 0

            lax.fori_loop(0, grp, drain, 0)
            return 0

        lax.fori_loop(0, n_chunks // grp, group, 0)
        plsc.subcore_barrier()

        pltpu.sync_copy(acc_sh.at[pl.ds(s * seg, seg)], stage_v)
        pltpu.sync_copy(stage_v, deg_hbm.at[pl.ds(c * NPAD + s * seg, seg)])

    return deg_kernel


def _make_scatter_kernel(ept, fd):
    n_chunks = ept // CH
    PH = 2
    pc = n_chunks // PH
    npairs = pc // 2
    seg = NPAD // NS

    @functools.partial(
        pl.kernel,
        out_type=jax.ShapeDtypeStruct((NC * NPAD, fd), jnp.float32),
        mesh=_MESH,
        compiler_params=_SC_PARAMS,
        scratch_types=[
            pltpu.VMEM((pc, CH), jnp.int32),
            pltpu.VMEM((pc, CH), jnp.int32),
            pltpu.VMEM((CH, fd), jnp.float32),
            pltpu.VMEM((CH, fd), jnp.float32),
            pltpu.VMEM_SHARED((NPAD, fd), jnp.float32),
            pltpu.SemaphoreType.DMA,
            pltpu.SemaphoreType.DMA,
            pltpu.SemaphoreType.DMA,
        ],
    )
    def scat_kernel(src_hbm, dst_hbm, feat_hbm, out_hbm,
                    si_v, di_v, rows0, rows1, acc_sh, sem_i, sem0, sem1):
        c = lax.axis_index("c")
        s = lax.axis_index("s")
        wid = c * NS + s

        si_dma = pltpu.async_copy(
            src_hbm.at[pl.ds(wid * n_chunks, pc)], si_v, sem_i)
        di_dma = pltpu.async_copy(
            dst_hbm.at[pl.ds(wid * n_chunks, pc)], di_v, sem_i)

        _zero_vmem_2d(rows0, CH, fd)
        nz = seg // CH

        def zc(i, _):
            pltpu.sync_copy(rows0, acc_sh.at[pl.ds(s * seg + i * CH, CH)])
            return 0

        lax.fori_loop(0, nz, zc, 0)
        plsc.subcore_barrier()

        for ph in range(PH):
            if ph > 0:
                pltpu.async_copy(
                    src_hbm.at[pl.ds(wid * n_chunks + ph * pc, pc)],
                    si_v, sem_i).wait()
                pltpu.async_copy(
                    dst_hbm.at[pl.ds(wid * n_chunks + ph * pc, pc)],
                    di_v, sem_i).wait()
            else:
                si_dma.wait()
                di_dma.wait()

            pltpu.async_copy(feat_hbm.at[si_v.at[0]], rows0, sem0)

            def pair(i, _):
                c0 = 2 * i
                pltpu.async_copy(feat_hbm.at[si_v.at[c0 + 1]], rows1, sem1)
                pltpu.make_async_copy(
                    feat_hbm.at[si_v.at[0]], rows0, sem0).wait()
                pltpu.sync_copy(rows0, acc_sh.at[di_v.at[c0]], add=True)

                @pl.when(i < npairs - 1)
                def _():
                    pltpu.async_copy(
                        feat_hbm.at[si_v.at[c0 + 2]], rows0, sem0)

                pltpu.make_async_copy(
                    feat_hbm.at[si_v.at[0]], rows1, sem1).wait()
                pltpu.sync_copy(rows1, acc_sh.at[di_v.at[c0 + 1]], add=True)
                return 0

            lax.fori_loop(0, npairs, pair, 0)

        plsc.subcore_barrier()

        def oc(i, _):
            pltpu.sync_copy(acc_sh.at[pl.ds(s * seg + i * CH, CH)], rows0)
            pltpu.sync_copy(
                rows0, out_hbm.at[pl.ds(c * NPAD + s * seg + i * CH, CH)])
            return 0

        lax.fori_loop(0, nz, oc, 0)

    return scat_kernel


def _tc_b(deg0_ref, deg1_ref, x_ref, w1_ref, hs_ref, dinv_ref):
    deg = deg0_ref[...] + deg1_ref[...] + 1.0
    dinv = lax.rsqrt(deg)
    h = jnp.dot(x_ref[...], w1_ref[...], preferred_element_type=jnp.float32)
    hs_ref[...] = h * dinv
    dinv_ref[...] = dinv


def _tc_d(acc0_ref, acc1_ref, hs_ref, dinv_ref, b1_ref, w2_ref, gs_ref):
    i = pl.program_id(0)
    dinv = dinv_ref[...]
    pre = acc0_ref[...] + acc1_ref[...] + hs_ref[...]
    o = pre * dinv + b1_ref[...]
    h1 = jnp.maximum(o, 0.0)
    g = jnp.dot(h1, w2_ref[...], preferred_element_type=jnp.float32)
    row = jax.lax.broadcasted_iota(jnp.int32, (BR, 1), 0) + i * BR
    gs_ref[...] = jnp.where(row < N, g * dinv, 0.0)


def _tc_f(a0_ref, a1_ref, gs_ref, dinv_ref, b2_ref, out_ref):
    o = (a0_ref[...] + a1_ref[...] + gs_ref[...]) * dinv_ref[...] + b2_ref[...]
    o8 = o[:, :8]
    m = jnp.max(o8, axis=1, keepdims=True)
    e = jnp.exp(o8 - m)
    lse = jnp.log(jnp.sum(e, axis=1, keepdims=True))
    out_ref[...] = o8 - m - lse


def kernel(x, edge_index, W1, b1, W2, b2):
    E = edge_index.shape[1]
    ept = ((E + NW * 2 * CH - 1) // (NW * 2 * CH)) * (2 * CH)
    epad = ept * NW
    pad = epad - E
    n_chunks = ept // CH

    src = jnp.concatenate(
        [edge_index[0], jnp.full((pad,), N, jnp.int32)]
    ).reshape(NW * n_chunks, CH)
    dst = jnp.concatenate(
        [edge_index[1], jnp.full((pad,), N, jnp.int32)]
    ).reshape(NW * n_chunks, CH)

    x_pad = jnp.pad(x, ((0, NPAD - N), (0, 0)))
    w2p = jnp.pad(W2, ((0, 0), (0, DO - W2.shape[1])))
    b1r = b1.reshape(1, D)
    b2r = jnp.pad(b2, (0, DO - b2.shape[0])).reshape(1, DO)

    deg = _make_deg_kernel(ept)(dst)
    deg0 = deg[:NPAD].reshape(NPAD, 1)
    deg1 = deg[NPAD:].reshape(NPAD, 1)

    grid = NPAD // BR
    hs, dinv = pl.pallas_call(
        _tc_b,
        grid=(grid,),
        in_specs=[
            pl.BlockSpec((BR, 1), lambda i: (i, 0)),
            pl.BlockSpec((BR, 1), lambda i: (i, 0)),
            pl.BlockSpec((BR, D), lambda i: (i, 0)),
            pl.BlockSpec((D, D), lambda i: (0, 0)),
        ],
        out_specs=[
            pl.BlockSpec((BR, D), lambda i: (i, 0)),
            pl.BlockSpec((BR, 1), lambda i: (i, 0)),
        ],
        out_shape=[
            jax.ShapeDtypeStruct((NPAD, D), jnp.float32),
            jax.ShapeDtypeStruct((NPAD, 1), jnp.float32),
        ],
    )(deg0, deg1, x_pad, W1)

    acc = _make_scatter_kernel(ept, D)(src, dst, hs)
    acc0 = acc[:NPAD]
    acc1 = acc[NPAD:]

    gs = pl.pallas_call(
        _tc_d,
        grid=(grid,),
        in_specs=[
            pl.BlockSpec((BR, D), lambda i: (i, 0)),
            pl.BlockSpec((BR, D), lambda i: (i, 0)),
            pl.BlockSpec((BR, D), lambda i: (i, 0)),
            pl.BlockSpec((BR, 1), lambda i: (i, 0)),
            pl.BlockSpec((1, D), lambda i: (0, 0)),
            pl.BlockSpec((D, DO), lambda i: (0, 0)),
        ],
        out_specs=pl.BlockSpec((BR, DO), lambda i: (i, 0)),
        out_shape=jax.ShapeDtypeStruct((NPAD, DO), jnp.float32),
    )(acc0, acc1, hs, dinv, b1r, w2p)

    acc2 = _make_scatter_kernel(ept, DO)(src, dst, gs)
    a20 = acc2[:NPAD]
    a21 = acc2[NPAD:]

    out = pl.pallas_call(
        _tc_f,
        grid=(grid,),
        in_specs=[
            pl.BlockSpec((BR, DO), lambda i: (i, 0)),
            pl.BlockSpec((BR, DO), lambda i: (i, 0)),
            pl.BlockSpec((BR, DO), lambda i: (i, 0)),
            pl.BlockSpec((BR, 1), lambda i: (i, 0)),
            pl.BlockSpec((1, DO), lambda i: (0, 0)),
        ],
        out_specs=pl.BlockSpec((BR, 8), lambda i: (i, 0)),
        out_shape=jax.ShapeDtypeStruct((NPAD, 8), jnp.float32),
    )(a20, a21, gs, dinv, b2r)

    return out[:N]
